# baseline (speedup 1.0000x reference)
import functools

import jax
import jax.numpy as jnp
from jax import lax
from jax.experimental import pallas as pl
from jax.experimental.pallas import tpu as pltpu
from jax.experimental.pallas import tpu_sc as plsc

_V = 1000000
_B = 4096
_L = 200
_D = 64
_NW = 32
_TCOLS_FULL = _V // 128
_TAIL_COLS = 1
_PROWS = _V // 2


def _repack_table(table_t, tail_pairs):
    mesh = plsc.VectorSubcoreMesh(core_axis_name="c", subcore_axis_name="s")

    @functools.partial(
        pl.kernel,
        mesh=mesh,
        compiler_params=pltpu.CompilerParams(needs_layout_passes=False),
        out_type=jax.ShapeDtypeStruct((_PROWS, 128), jnp.float32),
        scratch_types=[
            pltpu.VMEM((_D, 128), jnp.float32),
            pltpu.VMEM((_D, 128), jnp.float32),
            pltpu.SemaphoreType.DMA,
        ],
    )
    def k1(table_hbm, tail_hbm, scratch_hbm, inbuf, obuf, sem):
        wid = lax.axis_index("s") * 2 + lax.axis_index("c")
        rows16 = [lax.iota(jnp.int32, 16) + 16 * t for t in range(4)]

        def transpose_cols(ncols):
            for q in range(ncols // 2):
                for e in range(2):
                    cidx = jnp.full((16,), 2 * q + e, jnp.int32)
                    for t in range(4):
                        obuf[q, pl.ds(64 * e + 16 * t, 16)] = (
                            plsc.load_gather(inbuf, [rows16[t], cidx]))

        def body(jj, carry):
            j = wid + _NW * jj

            @pl.when(j < _TCOLS_FULL)
            def _():
                c0 = pl.multiple_of(128 * j, 128)
                pltpu.sync_copy(table_hbm.at[:, pl.ds(c0, 128)], inbuf)
                transpose_cols(128)
                pltpu.sync_copy(obuf, scratch_hbm.at[pl.ds(64 * j, 64)])

            return carry

        lax.fori_loop(0, _TCOLS_FULL // _NW + 1, body, 0)

        @pl.when(wid == _NW - 1)
        def _tail():
            pltpu.sync_copy(tail_hbm, obuf.at[pl.ds(0, 32)])
            pltpu.sync_copy(obuf.at[pl.ds(0, 32)],
                            scratch_hbm.at[pl.ds(64 * _TCOLS_FULL, 32)])

    return k1(table_t, tail_pairs)


def _gather_assemble(scratch, seq_t, pos_t):
    mesh = plsc.VectorSubcoreMesh(core_axis_name="c", subcore_axis_name="s")

    @functools.partial(
        pl.kernel,
        mesh=mesh,
        compiler_params=pltpu.CompilerParams(needs_layout_passes=False),
        out_type=jax.ShapeDtypeStruct((_L, _D, _B), jnp.float32),
        scratch_types=[
            pltpu.VMEM((128,), jnp.int32),
            pltpu.VMEM((_D, 200), jnp.float32),
            pltpu.VMEM((128, 128), jnp.float32),
            pltpu.VMEM((_D, 128), jnp.float32),
            pltpu.SemaphoreType.DMA,
        ],
    )
    def k2(scratch_hbm, seq_hbm, pos_hbm, out_hbm, ids_v, pos_v, grows_v,
           obuf, sem):
        wid = lax.axis_index("s") * 2 + lax.axis_index("c")
        b0 = pl.multiple_of(128 * wid, 128)
        pltpu.sync_copy(pos_hbm.at[:, pl.ds(0, 128)],
                        pos_v.at[:, pl.ds(0, 128)])
        pltpu.sync_copy(pos_hbm.at[:, pl.ds(128, 72)],
                        pos_v.at[:, pl.ds(128, 72)])
        rows16 = [lax.iota(jnp.int32, 16) + 16 * t for t in range(8)]

        def body(l, carry):
            pltpu.sync_copy(seq_hbm.at[l, pl.ds(b0, 128)], ids_v)
            pars = []
            for t in range(8):
                sl = pl.ds(16 * t, 16)
                idv = ids_v[sl]
                pars.append((idv & 1) * 64)
                ids_v[sl] = lax.shift_right_logical(idv, 1)
            pltpu.async_copy(scratch_hbm.at[ids_v], grows_v, sem).wait()
            lsplat = jnp.full((16,), 0, jnp.int32) + l
            for d in range(_D):
                splat = plsc.load_gather(
                    pos_v, [jnp.full((16,), d, jnp.int32), lsplat])
                for t in range(8):
                    v = plsc.load_gather(grows_v, [rows16[t], pars[t] + d])
                    obuf[d, pl.ds(16 * t, 16)] = v + splat
            pltpu.sync_copy(obuf, out_hbm.at[l, :, pl.ds(b0, 128)])
            return carry

        lax.fori_loop(0, _L, body, 0)

    return k2(scratch, seq_t, pos_t)


def kernel(seq, token_table, pos_table):
    table_t = token_table.T
    seq_t = seq.astype(jnp.int32).T
    pos_t = pos_table.T
    tail_pairs = token_table[128 * _TCOLS_FULL:].reshape(32, 128)
    scratch = _repack_table(table_t, tail_pairs)
    out_p = _gather_assemble(scratch, seq_t, pos_t)
    return out_p.transpose(2, 0, 1)


# speedup vs baseline: 1.1738x; 1.1738x over previous
import functools

import jax
import jax.numpy as jnp
from jax import lax
from jax.experimental import pallas as pl
from jax.experimental.pallas import tpu as pltpu
from jax.experimental.pallas import tpu_sc as plsc

_V = 1000000
_B = 4096
_L = 200
_D = 64
_NW = 32
_TCOLS_FULL = _V // 128
_PROWS = _V // 2


def _repack_table(table_t, tail_pairs):
    mesh = plsc.VectorSubcoreMesh(core_axis_name="c", subcore_axis_name="s")

    @functools.partial(
        pl.kernel,
        mesh=mesh,
        compiler_params=pltpu.CompilerParams(needs_layout_passes=False),
        out_type=jax.ShapeDtypeStruct((_PROWS, 128), jnp.float32),
        scratch_types=[
            pltpu.VMEM((2, _D, 128), jnp.float32),
            pltpu.VMEM((2, _D, 128), jnp.float32),
        ]
        + [pltpu.SemaphoreType.DMA] * 4,
    )
    def k1(table_hbm, tail_hbm, scratch_hbm, inbuf, obuf, *sems):
        isem = sems[:2]
        osem = sems[2:]
        wid = lax.axis_index("s") * 2 + lax.axis_index("c")
        rows16 = [lax.iota(jnp.int32, 16) + 16 * t for t in range(4)]

        def col_of(jj):
            return wid + _NW * jj

        def start_in(jj, p):
            j = col_of(jj)
            c0 = pl.multiple_of(128 * j, 128)
            pltpu.async_copy(table_hbm.at[:, pl.ds(c0, 128)], inbuf.at[p],
                             isem[p])

        def wait_in(p):
            pltpu.make_async_copy(table_hbm.at[:, pl.ds(0, 128)],
                                  inbuf.at[p], isem[p]).wait()

        def start_out(jj, p):
            j = col_of(jj)
            pltpu.async_copy(obuf.at[p], scratch_hbm.at[pl.ds(64 * j, 64)],
                             osem[p])

        def wait_out(p):
            pltpu.make_async_copy(obuf.at[p], scratch_hbm.at[pl.ds(0, 64)],
                                  osem[p]).wait()

        def transpose_cols(p):
            for q in range(64):
                for e in range(2):
                    cidx = jnp.full((16,), 2 * q + e, jnp.int32)
                    for t in range(4):
                        obuf[p, q, pl.ds(64 * e + 16 * t, 16)] = (
                            plsc.load_gather(inbuf.at[p], [rows16[t], cidx]))

        @pl.when(col_of(0) < _TCOLS_FULL)
        def _prime():
            start_in(0, 0)

        def outer(oi, carry):
            for p in range(2):
                jj = 2 * oi + p
                j = col_of(jj)
                jn = col_of(jj + 1)

                @pl.when(jn < _TCOLS_FULL)
                def _():
                    start_in(jj + 1, 1 - p)

                @pl.when(j < _TCOLS_FULL)
                def _():
                    wait_in(p)

                    @pl.when(jj >= 2)
                    def _():
                        wait_out(p)

                    transpose_cols(p)
                    start_out(jj, p)
            return carry

        lax.fori_loop(0, (_TCOLS_FULL // _NW + 2) // 2, outer, 0)

        for jj in (242, 243, 244):
            cond = ((col_of(jj) < _TCOLS_FULL)
                    & (col_of(jj + 2) >= _TCOLS_FULL))

            @pl.when(cond)
            def _():
                wait_out(jj % 2)

        @pl.when(wid == _NW - 1)
        def _tail():
            pltpu.sync_copy(tail_hbm, obuf.at[0, pl.ds(0, 32)])
            pltpu.sync_copy(obuf.at[0, pl.ds(0, 32)],
                            scratch_hbm.at[pl.ds(64 * _TCOLS_FULL, 32)])

    return k1(table_t, tail_pairs)


def _gather_assemble(scratch, seq_t, pos_t):
    mesh = plsc.VectorSubcoreMesh(core_axis_name="c", subcore_axis_name="s")

    @functools.partial(
        pl.kernel,
        mesh=mesh,
        compiler_params=pltpu.CompilerParams(needs_layout_passes=False),
        out_type=jax.ShapeDtypeStruct((_L, _D, _B), jnp.float32),
        scratch_types=[
            pltpu.VMEM((2, 128), jnp.int32),
            pltpu.VMEM((2, 128), jnp.int32),
            pltpu.VMEM((_D, 200), jnp.float32),
            pltpu.VMEM((2, 128, 128), jnp.float32),
            pltpu.VMEM((2, _D, 128), jnp.float32),
        ]
        + [pltpu.SemaphoreType.DMA] * 4,
    )
    def k2(scratch_hbm, seq_hbm, pos_hbm, out_hbm, ids_v, pidx_v, pos_v,
           grows_v, obuf, *sems):
        gsem = sems[:2]
        osem = sems[2:]
        wid = lax.axis_index("s") * 2 + lax.axis_index("c")
        b0 = pl.multiple_of(128 * wid, 128)
        pltpu.sync_copy(pos_hbm.at[:, pl.ds(0, 128)],
                        pos_v.at[:, pl.ds(0, 128)])
        pltpu.sync_copy(pos_hbm.at[:, pl.ds(128, 72)],
                        pos_v.at[:, pl.ds(128, 72)])
        rows16 = [lax.iota(jnp.int32, 16) + 16 * t for t in range(8)]

        def fetch(l, p):
            pltpu.sync_copy(seq_hbm.at[l, pl.ds(b0, 128)], ids_v.at[p])
            for t in range(8):
                sl = pl.ds(16 * t, 16)
                pidx_v[p, sl] = lax.shift_right_logical(ids_v[p, sl], 1)
            pltpu.async_copy(scratch_hbm.at[pidx_v.at[p]], grows_v.at[p],
                             gsem[p])

        def wait_gather(p):
            pltpu.make_async_copy(scratch_hbm.at[pidx_v.at[p]],
                                  grows_v.at[p], gsem[p]).wait()

        def start_out(l, p):
            pltpu.async_copy(obuf.at[p], out_hbm.at[l, :, pl.ds(b0, 128)],
                             osem[p])

        def wait_out(p):
            pltpu.make_async_copy(obuf.at[p], out_hbm.at[0, :, pl.ds(0, 128)],
                                  osem[p]).wait()

        fetch(0, 0)

        def outer(oi, carry):
            for p in range(2):
                l = 2 * oi + p

                @pl.when(l + 1 < _L)
                def _():
                    fetch(l + 1, 1 - p)

                wait_gather(p)

                @pl.when(l >= 2)
                def _():
                    wait_out(p)

                pars = []
                for t in range(8):
                    pars.append((ids_v[p, pl.ds(16 * t, 16)] & 1) * 64)
                lsplat = jnp.full((16,), 0, jnp.int32) + l
                for d in range(_D):
                    splat = plsc.load_gather(
                        pos_v, [jnp.full((16,), d, jnp.int32), lsplat])
                    for t in range(8):
                        v = plsc.load_gather(grows_v.at[p],
                                             [rows16[t], pars[t] + d])
                        obuf[p, d, pl.ds(16 * t, 16)] = v + splat
                start_out(l, p)
            return carry

        lax.fori_loop(0, _L // 2, outer, 0)
        wait_out(0)
        wait_out(1)

    return k2(scratch, seq_t, pos_t)


def kernel(seq, token_table, pos_table):
    table_t = token_table.T
    seq_t = seq.astype(jnp.int32).T
    pos_t = pos_table.T
    tail_pairs = token_table[128 * _TCOLS_FULL:].reshape(32, 128)
    scratch = _repack_table(table_t, tail_pairs)
    out_p = _gather_assemble(scratch, seq_t, pos_t)
    return out_p.transpose(2, 0, 1)


# speedup vs baseline: 2.9146x; 2.4830x over previous
import functools

import jax
import jax.numpy as jnp
from jax import lax
from jax.experimental import pallas as pl
from jax.experimental.pallas import tpu as pltpu
from jax.experimental.pallas import tpu_sc as plsc

_B = 4096
_L = 200
_D = 64
_NW = 32
_BROWS = _B // _NW
_CHUNK = 40
_NCH = _L // _CHUNK
_NBUF = 4


def _sc_embed(seq, token_table, pos_table):
    mesh = plsc.VectorSubcoreMesh(core_axis_name="c", subcore_axis_name="s")

    @functools.partial(
        pl.kernel,
        mesh=mesh,
        compiler_params=pltpu.CompilerParams(use_tc_tiling_on_sc=False),
        out_type=jax.ShapeDtypeStruct((_B, _L, _D), jnp.float32),
        scratch_types=[
            pltpu.VMEM((_BROWS, _L), jnp.int32),
            pltpu.VMEM((_L, _D), jnp.float32),
            pltpu.VMEM((_NBUF, _L, _D), jnp.float32),
        ]
        + [pltpu.SemaphoreType.DMA] * (2 * _NBUF),
    )
    def k(table_hbm, idx_hbm, pos_hbm, out_hbm, idx_v, pos_v, rows_v, *sems):
        gsems = sems[:_NBUF]
        wsems = sems[_NBUF:]
        wid = lax.axis_index("s") * 2 + lax.axis_index("c")
        base_b = wid * _BROWS
        pltpu.sync_copy(idx_hbm.at[pl.ds(base_b, _BROWS)], idx_v)
        pltpu.sync_copy(pos_hbm, pos_v)

        def start_gather(g, b):
            for c in range(_NCH):
                pltpu.async_copy(
                    table_hbm.at[idx_v.at[g, pl.ds(c * _CHUNK, _CHUNK)]],
                    rows_v.at[b, pl.ds(c * _CHUNK, _CHUNK)], gsems[b])

        def wait_gather(b):
            for c in range(_NCH):
                pltpu.make_async_copy(
                    table_hbm.at[idx_v.at[0, pl.ds(0, _CHUNK)]],
                    rows_v.at[b, pl.ds(c * _CHUNK, _CHUNK)], gsems[b]).wait()

        def start_write(g, b):
            pltpu.async_copy(rows_v.at[b], out_hbm.at[base_b + g], wsems[b])

        def wait_write(b):
            pltpu.make_async_copy(rows_v.at[b], out_hbm.at[0], wsems[b]).wait()

        def add_pos(b):
            @plsc.parallel_loop(0, _L, unroll=4)
            def _addrow(r):
                for j in range(_D // 16):
                    sl = pl.ds(j * 16, 16)
                    rows_v[b, r, sl] = rows_v[b, r, sl] + pos_v[r, sl]

        for b in range(_NBUF - 1):
            start_gather(b, b)

        def outer(oi, carry):
            for b in range(_NBUF):
                g = oi * _NBUF + b
                b3 = (b + _NBUF - 1) % _NBUF

                @pl.when(g >= 1)
                def _():
                    wait_write(b3)

                @pl.when(g + _NBUF - 1 < _BROWS)
                def _():
                    start_gather(g + _NBUF - 1, b3)

                wait_gather(b)
                add_pos(b)
                start_write(g, b)
            return carry

        lax.fori_loop(0, _BROWS // _NBUF, outer, 0)
        wait_write((_BROWS - 1) % _NBUF)

    return k(token_table, seq, pos_table)


def kernel(seq, token_table, pos_table):
    return _sc_embed(seq.astype(jnp.int32), token_table, pos_table)
